# Initial kernel scaffold; baseline (speedup 1.0000x reference)
import jax
import jax.numpy as jnp
from jax.experimental import pallas as pl


def kernel(x, edge_index, edge_attr, W_node, b_node, W_edge, b_edge, W_fc, b_fc):
    raise NotImplementedError("write your pallas kernel here")


# speedup vs baseline: 144.4854x; 144.4854x over previous
import functools
import jax
import jax.numpy as jnp
from jax import lax
from jax.experimental import pallas as pl
from jax.experimental.pallas import tpu as pltpu
from jax.experimental.pallas import tpu_sc as plsc

N = 10000
E = 320000
NC, NS, L = 2, 16, 16
NW = NC * NS
EPW = E // NW
HB = 640


HBL = HB * L
DW = 128


def _hist_body(dst_hbm, zeros_hbm, out_hbm, dst_v, ones_v, shared_ref):
    c = lax.axis_index("c")
    s = lax.axis_index("s")
    wid = c * NS + s

    ones16 = jnp.ones((L,), jnp.float32)
    for k in range(L):
        for j in range(DW // L):
            ones_v[k, pl.ds(j * L, L)] = ones16

    @pl.when(s == 0)
    def _():
        pltpu.sync_copy(zeros_hbm, shared_ref)

    pltpu.sync_copy(dst_hbm.at[pl.ds(wid * EPW, EPW)], dst_v)
    plsc.subcore_barrier()

    def body(i, _):
        idx = dst_v[pl.ds(i * L, L)]
        pltpu.sync_copy(ones_v, shared_ref.at[idx], add=True)
        return 0

    lax.fori_loop(0, EPW // L, body, 0)
    plsc.subcore_barrier()

    @pl.when(s == 0)
    def _():
        pltpu.sync_copy(shared_ref, out_hbm.at[pl.ds(c * HBL, HBL)])


@functools.partial(jax.jit, static_argnames=())
def _degree_hist(dst_i32):
    mesh = plsc.VectorSubcoreMesh(core_axis_name="c", subcore_axis_name="s")
    f = pl.kernel(
        _hist_body,
        out_type=jax.ShapeDtypeStruct((NC * HBL, DW), jnp.float32),
        mesh=mesh,
        scratch_types=[
            pltpu.VMEM((EPW,), jnp.int32),
            pltpu.VMEM((L, DW), jnp.float32),
            pltpu.VMEM_SHARED((HBL, DW), jnp.float32),
        ],
    )
    return f(dst_i32, jnp.zeros((HBL, DW), jnp.float32))


D2 = 128
NB = 5
CPT = EPW // L


def _seg_body(hs2_hbm, src_hbm, dst_hbm, zeros_hbm, out_hbm,
              src_v, dst_v, gsem, ssem, shared_ref, *rows_v):
    c = lax.axis_index("c")
    s = lax.axis_index("s")
    wid = c * NS + s

    @pl.when(jnp.logical_and(s == 0, c == 0))
    def _():
        pltpu.sync_copy(hs2_hbm, shared_ref)

    @pl.when(jnp.logical_and(s == 0, c == 1))
    def _():
        pltpu.sync_copy(zeros_hbm, shared_ref)

    pltpu.sync_copy(src_hbm.at[pl.ds(wid * EPW, EPW)], src_v)
    pltpu.sync_copy(dst_hbm.at[pl.ds(wid * EPW, EPW)], dst_v)
    plsc.subcore_barrier()

    def round_(o, _):
        descs = []
        for b in range(NB):
            idx_ref = src_v.at[pl.ds((o * NB + b) * L, L)]
            descs.append(pltpu.async_copy(hs2_hbm.at[idx_ref], rows_v[b], gsem))
        for d in descs:
            d.wait()
        descs = []
        for b in range(NB):
            idx = dst_v[pl.ds((o * NB + b) * L, L)]
            descs.append(pltpu.async_copy(rows_v[b], shared_ref.at[idx],
                                          ssem, add=True))
        for d in descs:
            d.wait()
        return 0

    lax.fori_loop(0, CPT // NB, round_, 0)
    plsc.subcore_barrier()

    @pl.when(s == 0)
    def _():
        pltpu.sync_copy(shared_ref, out_hbm.at[pl.ds(c * N, N)])


@functools.partial(jax.jit, static_argnames=())
def _seg_sum(hs2, src_i32, dst_i32):
    mesh = plsc.VectorSubcoreMesh(core_axis_name="c", subcore_axis_name="s")
    f = pl.kernel(
        _seg_body,
        out_type=jax.ShapeDtypeStruct((NC * N, D2), jnp.float32),
        mesh=mesh,
        scratch_types=[
            pltpu.VMEM((EPW,), jnp.int32),
            pltpu.VMEM((EPW,), jnp.int32),
            pltpu.SemaphoreType.DMA,
            pltpu.SemaphoreType.DMA,
            pltpu.VMEM_SHARED((N, D2), jnp.float32),
        ] + [pltpu.VMEM((L, D2), jnp.float32) for _ in range(NB)],
    )
    return f(hs2, src_i32, dst_i32, jnp.zeros((N, D2), jnp.float32))


def kernel(x, edge_index, edge_attr, W_node, b_node, W_edge, b_edge, W_fc, b_fc):
    src = edge_index[0].astype(jnp.int32)
    dst = edge_index[1].astype(jnp.int32)

    hist2 = _degree_hist(dst)
    counts = (hist2[:HBL, 0] + hist2[HBL:, 0])[:N]
    deg = counts + 1.0
    a = lax.rsqrt(deg)

    h = x @ W_node
    he = edge_attr @ W_edge
    hs2 = jnp.concatenate([h * a[:, None], he[:N] * a[:, None]], axis=1)

    s2p = _seg_sum(hs2, src, dst)
    s2_plus_hs2 = s2p[:N] + s2p[N:]

    out2 = a[:, None] * s2_plus_hs2 + jnp.concatenate([b_node, b_edge])
    out2 = jax.nn.relu(out2)
    col = jnp.sum(out2, axis=0)
    tail = jnp.sum(jax.nn.relu(he[N:] + b_edge), axis=0)
    node_p = col[:64] / N
    edge_p = (col[64:] + tail) / E
    z = jnp.concatenate([node_p, edge_p])[None, :]
    logits = z @ W_fc + b_fc
    return jax.nn.softmax(logits, axis=1)
